# baseline (speedup 1.0000x reference)
import functools

import jax
import jax.numpy as jnp
from jax import lax
from jax.experimental import pallas as pl
from jax.experimental.pallas import tpu as pltpu
from jax.experimental.pallas import tpu_sc as plsc

BATCH = 4096
HIST = 200
HPAD = 224
HALF = HPAD // 2
EMB = 64
NLANE = 16
NWORKERS = 32
ROWS = BATCH // NWORKERS


def _body(x_hbm, m_hbm, pos_hbm, neg_hbm, tab_hbm, pos_out, neg_out,
          idx_v, msk_v, g_v, pidx_v, nidx_v, prow_v, nrow_v,
          ps_v, ns_v, idxa_v, idxb_v, sem_a, sem_b, sem_s):
    wid = lax.axis_index("c") * 16 + lax.axis_index("s")
    base = wid * ROWS

    pltpu.sync_copy(x_hbm.at[pl.ds(base, ROWS)], idx_v)
    pltpu.sync_copy(m_hbm.at[pl.ds(base, ROWS)], msk_v)
    pltpu.sync_copy(pos_hbm.at[pl.ds(base, ROWS)], pidx_v)
    pltpu.sync_copy(neg_hbm.at[pl.ds(base, ROWS)], nidx_v)

    cp = pltpu.async_copy(tab_hbm.at[pidx_v], prow_v, sem_a)
    cn = pltpu.async_copy(tab_hbm.at[nidx_v], nrow_v, sem_b)
    cp.wait()
    cn.wait()

    def mask_row(r, _):
        for h in range(2):
            for k in range(HALF // NLANE):
                sl = (r, h, pl.ds(k * NLANE, NLANE))
                idx_v[sl] = idx_v[sl] * msk_v[sl]
        return 0

    lax.fori_loop(0, ROWS, mask_row, 0)

    lanes = lax.iota(jnp.int32, NLANE)
    zero = jnp.zeros((NLANE,), jnp.float32)

    def one_row(r):
        for k in range(HALF // NLANE):
            sl = pl.ds(k * NLANE, NLANE)
            idxa_v[sl] = idx_v[r, 0, sl]
            idxb_v[sl] = idx_v[r, 1, sl]
        ca = pltpu.async_copy(tab_hbm.at[idxa_v], g_v.at[0], sem_a)
        cb = pltpu.async_copy(tab_hbm.at[idxb_v], g_v.at[1], sem_b)
        ca.wait()
        cb.wait()

        def acc_step(l, accs):
            out = []
            for h in range(2):
                for j in range(4):
                    out.append(accs[h * 4 + j] +
                               g_v[h, l, pl.ds(j * NLANE, NLANE)])
            return tuple(out)

        accs = lax.fori_loop(0, HALF, acc_step, (zero,) * 8)
        acc = [accs[j] + accs[4 + j] for j in range(4)]

        cvec = zero
        for h in range(2):
            for k in range(HALF // NLANE):
                cvec = cvec + msk_v[r, h, pl.ds(k * NLANE, NLANE)].astype(
                    jnp.float32)
        cnt = jnp.sum(cvec)

        pdot = zero
        ndot = zero
        for j in range(4):
            pdot = pdot + acc[j] * prow_v[r, pl.ds(j * NLANE, NLANE)]
            ndot = ndot + acc[j] * nrow_v[r, pl.ds(j * NLANE, NLANE)]
        return jnp.sum(pdot), jnp.sum(ndot), cnt

    def do_group(g, _):
        svp = zero
        svn = zero
        svc = zero
        for u in range(NLANE):
            sp, sn, cnt = one_row(g * NLANE + u)
            svp = jnp.where(lanes == u, sp, svp)
            svn = jnp.where(lanes == u, sn, svn)
            svc = jnp.where(lanes == u, cnt, svc)
        invv = 1.0 / jnp.maximum(svc, 1.0)
        off = pl.multiple_of(g * NLANE, NLANE)
        ps_v[pl.ds(off, NLANE)] = svp * invv
        ns_v[pl.ds(off, NLANE)] = svn * invv
        return 0

    lax.fori_loop(0, ROWS // NLANE, do_group, 0)

    pltpu.sync_copy(ps_v, pos_out.at[pl.ds(base, ROWS)])
    pltpu.sync_copy(ns_v, neg_out.at[pl.ds(base, ROWS)])


@jax.jit
def _run(x3, m3, pos_items, neg_items, item_emb):
    mesh = plsc.VectorSubcoreMesh(core_axis_name="c", subcore_axis_name="s",
                                  num_cores=2, num_subcores=16)
    f = pl.kernel(
        _body,
        out_type=(
            jax.ShapeDtypeStruct((BATCH,), jnp.float32),
            jax.ShapeDtypeStruct((BATCH,), jnp.float32),
        ),
        mesh=mesh,
        compiler_params=pltpu.CompilerParams(needs_layout_passes=False,
                                             use_tc_tiling_on_sc=False),
        scratch_types=[
            pltpu.VMEM((ROWS, 2, HALF), jnp.int32),
            pltpu.VMEM((ROWS, 2, HALF), jnp.int32),
            pltpu.VMEM((2, HALF, EMB), jnp.float32),
            pltpu.VMEM((ROWS,), jnp.int32),
            pltpu.VMEM((ROWS,), jnp.int32),
            pltpu.VMEM((ROWS, EMB), jnp.float32),
            pltpu.VMEM((ROWS, EMB), jnp.float32),
            pltpu.VMEM((ROWS,), jnp.float32),
            pltpu.VMEM((ROWS,), jnp.float32),
            pltpu.VMEM((HALF,), jnp.int32),
            pltpu.VMEM((HALF,), jnp.int32),
            pltpu.SemaphoreType.DMA,
            pltpu.SemaphoreType.DMA,
            pltpu.SemaphoreType.DMA,
        ],
    )
    return f(x3, m3, pos_items, neg_items, item_emb)


def kernel(x_pad, mask, pos_items, neg_items, item_emb):
    x = x_pad.astype(jnp.int32)
    m = mask.astype(jnp.int32)
    pad = HPAD - HIST
    x3 = jnp.pad(x, ((0, 0), (0, pad))).reshape(BATCH, 2, HALF)
    m3 = jnp.pad(m, ((0, 0), (0, pad))).reshape(BATCH, 2, HALF)
    pos_score, neg_score = _run(x3, m3, pos_items.astype(jnp.int32),
                                neg_items.astype(jnp.int32), item_emb)
    return (pos_score, neg_score)
